# Initial kernel scaffold; baseline (speedup 1.0000x reference)
import jax
import jax.numpy as jnp
from jax.experimental import pallas as pl


def kernel(x, table):
    raise NotImplementedError("write your pallas kernel here")


# speedup vs baseline: 1.3076x; 1.3076x over previous
import functools
import math

import jax
import jax.numpy as jnp
from jax import lax
from jax.experimental import pallas as pl
from jax.experimental.pallas import tpu as pltpu
from jax.experimental.pallas import tpu_sc as plsc

D = 32
L = 16
NC, NS = 2, 16
NW = NC * NS

B = 4096 * 200
B_PER_W = B // NW
CHUNK = 1600
N_CHUNKS = B_PER_W // CHUNK

_SCALE = math.sqrt(float(D))


def _body(x_hbm, table_hbm, out_hbm, idx_v, rows_v, sem):
    wid = lax.axis_index("s") * NC + lax.axis_index("c")
    base = wid * B_PER_W

    def chunk_step(c, carry):
        off = base + c * CHUNK
        pltpu.sync_copy(x_hbm.at[pl.ds(off, CHUNK)], idx_v)
        pltpu.async_copy(table_hbm.at[idx_v], rows_v, sem).wait()

        def scale_row(r, carry2):
            v0 = rows_v[r, pl.ds(0, L)]
            rows_v[r, pl.ds(0, L)] = v0 * _SCALE
            v1 = rows_v[r, pl.ds(L, L)]
            rows_v[r, pl.ds(L, L)] = v1 * _SCALE
            return carry2

        lax.fori_loop(0, CHUNK, scale_row, 0)
        pltpu.sync_copy(rows_v, out_hbm.at[pl.ds(off, CHUNK)])
        return carry

    lax.fori_loop(0, N_CHUNKS, chunk_step, 0)


@functools.partial(
    pl.kernel,
    mesh=plsc.VectorSubcoreMesh(core_axis_name="c", subcore_axis_name="s"),
    out_type=jax.ShapeDtypeStruct((B, D), jnp.float32),
    scratch_types=[
        pltpu.VMEM((CHUNK,), jnp.int32),
        pltpu.VMEM((CHUNK, D), jnp.float32),
        pltpu.SemaphoreType.DMA,
    ],
    compiler_params=pltpu.CompilerParams(use_tc_tiling_on_sc=False),
)
def _gather_scale(x_hbm, table_hbm, out_hbm, idx_v, rows_v, sem):
    _body(x_hbm, table_hbm, out_hbm, idx_v, rows_v, sem)


def kernel(x, table):
    out = _gather_scale(x.reshape(B), table)
    return out.reshape(x.shape[0], x.shape[1], D)


# speedup vs baseline: 1.4122x; 1.0800x over previous
import functools
import math

import jax
import jax.numpy as jnp
from jax import lax
from jax.experimental import pallas as pl
from jax.experimental.pallas import tpu as pltpu
from jax.experimental.pallas import tpu_sc as plsc

D = 32
L = 16
NC, NS = 2, 16
NW = NC * NS

B = 4096 * 200
B_PER_W = B // NW
NBUF = 4
CHUNK = 640
N_CHUNKS = B_PER_W // CHUNK

_SCALE = math.sqrt(float(D))


def _scale_chunk(rows):
    def scale_row(r, carry):
        v0 = rows[r, pl.ds(0, L)]
        rows[r, pl.ds(0, L)] = v0 * _SCALE
        v1 = rows[r, pl.ds(L, L)]
        rows[r, pl.ds(L, L)] = v1 * _SCALE
        return carry

    lax.fori_loop(0, CHUNK, scale_row, 0)


def _body(x_hbm, table_hbm, out_hbm, idx_v, rows, gsems, ssems):
    wid = lax.axis_index("s") * NC + lax.axis_index("c")
    base = wid * B_PER_W
    pltpu.sync_copy(x_hbm.at[pl.ds(base, B_PER_W)], idx_v)

    def gather_start(c, b):
        pltpu.async_copy(
            table_hbm.at[idx_v.at[pl.ds(c * CHUNK, CHUNK)]], rows[b], gsems[b])

    def gather_wait(c, b):
        pltpu.make_async_copy(
            table_hbm.at[idx_v.at[pl.ds(c * CHUNK, CHUNK)]], rows[b],
            gsems[b]).wait()

    def store_start(c, b):
        pltpu.async_copy(
            rows[b], out_hbm.at[pl.ds(base + c * CHUNK, CHUNK)], ssems[b])

    def store_wait(c, b):
        pltpu.make_async_copy(
            rows[b], out_hbm.at[pl.ds(base + c * CHUNK, CHUNK)],
            ssems[b]).wait()

    for b in range(NBUF - 1):
        gather_start(b, b)

    def outer(g, carry):
        for b in range(NBUF):
            c = g + b
            bp = (b - 1) % NBUF
            gather_wait(c, b)
            _scale_chunk(rows[b])
            store_start(c, b)
            if b == 0:
                @pl.when(g > 0)
                def _():
                    store_wait(c - 1, bp)
            else:
                store_wait(c - 1, bp)

            @pl.when(c + NBUF - 1 < N_CHUNKS)
            def _():
                gather_start(c + NBUF - 1, bp)
        return carry

    lax.fori_loop(0, N_CHUNKS // NBUF, lambda i, cr: outer(i * NBUF, cr), 0)
    store_wait(N_CHUNKS - 1, (N_CHUNKS - 1) % NBUF)


@functools.partial(
    pl.kernel,
    mesh=plsc.VectorSubcoreMesh(core_axis_name="c", subcore_axis_name="s"),
    out_type=jax.ShapeDtypeStruct((B, D), jnp.float32),
    scratch_types=[
        pltpu.VMEM((B_PER_W,), jnp.int32),
        [pltpu.VMEM((CHUNK, D), jnp.float32) for _ in range(NBUF)],
        [pltpu.SemaphoreType.DMA for _ in range(NBUF)],
        [pltpu.SemaphoreType.DMA for _ in range(NBUF)],
    ],
    compiler_params=pltpu.CompilerParams(use_tc_tiling_on_sc=False),
)
def _gather_scale(x_hbm, table_hbm, out_hbm, idx_v, rows, gsems, ssems):
    _body(x_hbm, table_hbm, out_hbm, idx_v, rows, gsems, ssems)


def kernel(x, table):
    out = _gather_scale(x.reshape(B), table)
    return out.reshape(x.shape[0], x.shape[1], D)
